# Initial kernel scaffold; baseline (speedup 1.0000x reference)
import functools

import jax
import jax.numpy as jnp
from jax import lax
from jax.experimental import pallas as pl
from jax.experimental.pallas import tpu as pltpu
from jax.experimental.pallas import tpu_sc as plsc

N = 10000
E = 320000
D = 128
DH = 64
DOUT = 64
ALPHA = 0.1

EB = 128
NBLK_PAD = 2560
E_PAD = NBLK_PAD * EB
TB = NBLK_PAD // 16
DB = NBLK_PAD // 32
NPADROWS = 128
NT = N + NPADROWS
DEGT = 10240
RPT = N // 16
NBUF = 4

_MESH = plsc.VectorSubcoreMesh(
    core_axis_name="c", subcore_axis_name="s", num_cores=2, num_subcores=16
)


def _mm_relu_body(x_ref, w_ref, b_ref, o_ref):
    acc = jnp.dot(x_ref[...], w_ref[...], preferred_element_type=jnp.float32)
    o_ref[...] = jnp.maximum(acc + b_ref[...], 0.0)


def _scale_body(h_ref, d_ref, t_ref):
    dinv = lax.rsqrt(d_ref[0, :] + d_ref[1, :] + 1.0)
    t = h_ref[...] * dinv[:, None]
    t_ref[0] = t[:, :DH]
    t_ref[1] = t[:, DH:]


def _layer_body(s_ref, h0_ref, d_ref, w_ref, t_ref):
    dinv = lax.rsqrt(d_ref[0, :] + d_ref[1, :] + 1.0)
    scat = jnp.concatenate([s_ref[0], s_ref[1]], axis=1)
    u = (1.0 - ALPHA) * (scat * dinv[:, None]) + ALPHA * h0_ref[...]
    h2 = jnp.dot(u, w_ref[...], preferred_element_type=jnp.float32)
    t2 = h2 * dinv[:, None]
    t_ref[0] = t2[:, :DH]
    t_ref[1] = t2[:, DH:]


def _final_body(s_ref, h0_ref, d_ref, w_ref, w2_ref, b2_ref, h_ref, lg_ref):
    dinv = lax.rsqrt(d_ref[0, :] + d_ref[1, :] + 1.0)
    scat = jnp.concatenate([s_ref[0], s_ref[1]], axis=1)
    u = (1.0 - ALPHA) * (scat * dinv[:, None]) + ALPHA * h0_ref[...]
    h3 = jnp.dot(u, w_ref[...], preferred_element_type=jnp.float32)
    h_ref[...] = h3
    lg = jnp.dot(h3, w2_ref[...], preferred_element_type=jnp.float32)
    lg_ref[...] = lg + b2_ref[...]


_BM = 1000
_GRID = (N // _BM,)


def _row_spec(w):
    return pl.BlockSpec((_BM, w), lambda i: (i, 0))


def _full_spec(r, c):
    return pl.BlockSpec((r, c), lambda i: (0, 0))


_D_SPEC = pl.BlockSpec((2, _BM), lambda i: (0, i))
_S_SPEC = pl.BlockSpec((2, _BM, DH), lambda i: (0, i, 0))


def _deg_body(dst_hbm, out_hbm, deg_sp, idx_v, ones_v, z_v, sem):
    c = lax.axis_index("c")
    s = lax.axis_index("s")
    for i in range(EB // 16):
        ones_v[pl.ds(i * 16, 16)] = jnp.ones((16,), jnp.float32)
    for i in range(640 // 16):
        z_v[pl.ds(i * 16, 16)] = jnp.zeros((16,), jnp.float32)
    pltpu.sync_copy(z_v, deg_sp.at[pl.ds(s * 640, 640)])
    plsc.subcore_barrier()
    blk0 = (c * 16 + s) * DB
    pltpu.async_copy(dst_hbm.at[pl.ds(blk0, DB), :], idx_v, sem).wait()
    for j in range(DB):
        pltpu.sync_copy(ones_v, deg_sp.at[idx_v.at[j]], add=True)
    plsc.subcore_barrier()
    pltpu.sync_copy(deg_sp.at[pl.ds(s * 640, 640)],
                    out_hbm.at[c, pl.ds(s * 640, 640)])


_deg_kernel = functools.partial(
    pl.kernel,
    out_type=jax.ShapeDtypeStruct((2, DEGT), jnp.float32),
    mesh=_MESH,
    scratch_types=[
        pltpu.VMEM_SHARED((DEGT,), jnp.float32),
        pltpu.VMEM((DB, EB), jnp.int32),
        pltpu.VMEM((EB,), jnp.float32),
        pltpu.VMEM((640,), jnp.float32),
        pltpu.SemaphoreType.DMA,
    ],
)(_deg_body)


def _prop_body(t_hbm, src_hbm, dst_hbm, out_hbm, t_sp, s_sp, si_v, di_v,
               rows_v, z_v, sem0, g0, g1, g2, g3, s0, s1, s2, s3):
    gs = (g0, g1, g2, g3)
    ss = (s0, s1, s2, s3)
    c = lax.axis_index("c")
    s = lax.axis_index("s")
    for r in range(8):
        for k in range(DH // 16):
            z_v[r, pl.ds(k * 16, 16)] = jnp.zeros((16,), jnp.float32)
    r0 = s * RPT
    pltpu.async_copy(t_hbm.at[c, pl.ds(r0, RPT), :],
                     t_sp.at[pl.ds(r0, RPT)], sem0).wait()
    pltpu.async_copy(t_hbm.at[c, pl.ds(r0, RPT), :],
                     s_sp.at[pl.ds(r0, RPT)], sem0).wait()
    pltpu.sync_copy(z_v, t_sp.at[pl.ds(N + s * 8, 8)])
    pltpu.sync_copy(z_v, s_sp.at[pl.ds(N + s * 8, 8)])
    blk0 = s * TB
    pltpu.async_copy(src_hbm.at[pl.ds(blk0, TB), :], si_v, sem0).wait()
    pltpu.async_copy(dst_hbm.at[pl.ds(blk0, TB), :], di_v, sem0).wait()
    plsc.subcore_barrier()

    for b in range(NBUF):
        pltpu.async_copy(t_sp.at[si_v.at[b]], rows_v.at[b], gs[b])

    def outer(g, carry):
        for b in range(NBUF):
            j = g * NBUF + b
            pltpu.make_async_copy(
                t_sp.at[si_v.at[j]], rows_v.at[b], gs[b]).wait()
            pltpu.async_copy(
                rows_v.at[b], s_sp.at[di_v.at[j]], ss[b], add=True).wait()
            pltpu.async_copy(
                t_sp.at[si_v.at[j + NBUF]], rows_v.at[b], gs[b])
        return carry

    lax.fori_loop(0, TB // NBUF - 1, outer, 0)
    for b in range(NBUF):
        j = TB - NBUF + b
        pltpu.make_async_copy(t_sp.at[si_v.at[j]], rows_v.at[b], gs[b]).wait()
        pltpu.async_copy(
            rows_v.at[b], s_sp.at[di_v.at[j]], ss[b], add=True).wait()

    plsc.subcore_barrier()
    pltpu.async_copy(s_sp.at[pl.ds(r0, RPT)],
                     out_hbm.at[c, pl.ds(r0, RPT), :], sem0).wait()


_prop_kernel = functools.partial(
    pl.kernel,
    out_type=jax.ShapeDtypeStruct((2, N, DH), jnp.float32),
    mesh=_MESH,
    scratch_types=[
        pltpu.VMEM_SHARED((NT, DH), jnp.float32),
        pltpu.VMEM_SHARED((NT, DH), jnp.float32),
        pltpu.VMEM((TB, EB), jnp.int32),
        pltpu.VMEM((TB, EB), jnp.int32),
        pltpu.VMEM((NBUF, EB, DH), jnp.float32),
        pltpu.VMEM((8, DH), jnp.float32),
        pltpu.SemaphoreType.DMA,
        pltpu.SemaphoreType.DMA,
        pltpu.SemaphoreType.DMA,
        pltpu.SemaphoreType.DMA,
        pltpu.SemaphoreType.DMA,
        pltpu.SemaphoreType.DMA,
        pltpu.SemaphoreType.DMA,
        pltpu.SemaphoreType.DMA,
        pltpu.SemaphoreType.DMA,
    ],
)(_prop_body)


def kernel(x, edge_index, W1, b1, cw1, cw2, W2, b2):
    src = edge_index[0].astype(jnp.int32)
    dst = edge_index[1].astype(jnp.int32)
    padi = N + (lax.iota(jnp.int32, E_PAD - E) % NPADROWS)
    srcp = jnp.concatenate([src, padi]).reshape(NBLK_PAD, EB)
    dstp = jnp.concatenate([dst, padi]).reshape(NBLK_PAD, EB)

    h = pl.pallas_call(
        _mm_relu_body,
        grid=_GRID,
        in_specs=[_row_spec(D), _full_spec(D, D), _full_spec(1, D)],
        out_specs=_row_spec(D),
        out_shape=jax.ShapeDtypeStruct((N, D), jnp.float32),
    )(x, W1, b1.reshape(1, D))

    degp = _deg_kernel(dstp)
    deg2 = degp[:, :N]

    t1 = pl.pallas_call(
        _scale_body,
        grid=_GRID,
        in_specs=[_row_spec(D), _D_SPEC],
        out_specs=_S_SPEC,
        out_shape=jax.ShapeDtypeStruct((2, N, DH), jnp.float32),
    )(h, deg2)

    s1 = _prop_kernel(t1, srcp, dstp)

    t2 = pl.pallas_call(
        _layer_body,
        grid=_GRID,
        in_specs=[_S_SPEC, _row_spec(D), _D_SPEC, _full_spec(D, D)],
        out_specs=_S_SPEC,
        out_shape=jax.ShapeDtypeStruct((2, N, DH), jnp.float32),
    )(s1, h, deg2, cw1)

    s2 = _prop_kernel(t2, srcp, dstp)

    h_out, logits = pl.pallas_call(
        _final_body,
        grid=_GRID,
        in_specs=[_S_SPEC, _row_spec(D), _D_SPEC, _full_spec(D, D),
                  _full_spec(D, DOUT), _full_spec(1, DOUT)],
        out_specs=[_row_spec(D), _row_spec(DOUT)],
        out_shape=[jax.ShapeDtypeStruct((N, D), jnp.float32),
                   jax.ShapeDtypeStruct((N, DOUT), jnp.float32)],
    )(s2, h, deg2, cw2, W2, b2.reshape(1, DOUT))

    return (h_out, logits)


# speedup vs baseline: 32.2359x; 32.2359x over previous
import functools

import jax
import jax.numpy as jnp
from jax import lax
from jax.experimental import pallas as pl
from jax.experimental.pallas import tpu as pltpu
from jax.experimental.pallas import tpu_sc as plsc

N = 10000
E = 320000
D = 128
DH = 64
DOUT = 64
ALPHA = 0.1

EB = 128
NBLK = E // EB
PB = 156
NBUF = 4
TPB = PB // NBUF
TPB15 = (NBLK - 15 * PB) // NBUF
DB = 78
DB31 = NBLK - 31 * DB
DEGT = 10240
RPT = N // 16

@functools.lru_cache(maxsize=None)
def _sc_mesh():
    return plsc.VectorSubcoreMesh(
        core_axis_name="c", subcore_axis_name="s", num_cores=2, num_subcores=16
    )


def _mm_relu_body(x_ref, w_ref, b_ref, o_ref):
    acc = jnp.dot(x_ref[...], w_ref[...], preferred_element_type=jnp.float32)
    o_ref[...] = jnp.maximum(acc + b_ref[...], 0.0)


def _scale_body(h_ref, d_ref, t_ref):
    dinv = lax.rsqrt(d_ref[:, 0] + d_ref[:, 1] + 1.0)
    t = h_ref[...] * dinv[:, None]
    t_ref[0] = t[:, :DH]
    t_ref[1] = t[:, DH:]


def _layer_body(s_ref, h0_ref, d_ref, w_ref, t_ref):
    dinv = lax.rsqrt(d_ref[:, 0] + d_ref[:, 1] + 1.0)
    scat = jnp.concatenate([s_ref[0], s_ref[1]], axis=1)
    u = (1.0 - ALPHA) * (scat * dinv[:, None]) + ALPHA * h0_ref[...]
    h2 = jnp.dot(u, w_ref[...], preferred_element_type=jnp.float32)
    t2 = h2 * dinv[:, None]
    t_ref[0] = t2[:, :DH]
    t_ref[1] = t2[:, DH:]


def _final_body(s_ref, h0_ref, d_ref, w_ref, w2_ref, b2_ref, h_ref, lg_ref):
    dinv = lax.rsqrt(d_ref[:, 0] + d_ref[:, 1] + 1.0)
    scat = jnp.concatenate([s_ref[0], s_ref[1]], axis=1)
    u = (1.0 - ALPHA) * (scat * dinv[:, None]) + ALPHA * h0_ref[...]
    h3 = jnp.dot(u, w_ref[...], preferred_element_type=jnp.float32)
    h_ref[...] = h3
    lg = jnp.dot(h3, w2_ref[...], preferred_element_type=jnp.float32)
    lg_ref[...] = lg + b2_ref[...]


_BM = 1000
_GRID = (N // _BM,)


def _row_spec(w):
    return pl.BlockSpec((_BM, w), lambda i: (i, 0))


def _full_spec(r, c):
    return pl.BlockSpec((r, c), lambda i: (0, 0))


_D_SPEC = pl.BlockSpec((_BM, 2), lambda i: (i, 0))
_S_SPEC = pl.BlockSpec((2, _BM, DH), lambda i: (0, i, 0))


def _deg_body(dst_hbm, ones_hbm, zeros_hbm, out_hbm, deg_sp, idx_v, ones_v,
              sem):
    c = lax.axis_index("c")
    s = lax.axis_index("s")
    wid = c * 16 + s
    pltpu.async_copy(ones_hbm, ones_v, sem).wait()
    pltpu.sync_copy(zeros_hbm, deg_sp.at[pl.ds(s * 640, 640)])
    plsc.subcore_barrier()
    base = wid * DB
    pltpu.async_copy(dst_hbm.at[pl.ds(base, DB), :],
                     idx_v.at[pl.ds(0, DB)], sem).wait()

    @pl.when(wid == 31)
    def _():
        pltpu.async_copy(dst_hbm.at[pl.ds(32 * DB, DB31 - DB), :],
                         idx_v.at[pl.ds(DB, DB31 - DB)], sem).wait()

    cnt = jnp.where(wid == 31, DB31, DB)

    def _hist(j, carry):
        pltpu.sync_copy(ones_v, deg_sp.at[idx_v.at[j]], add=True)
        return carry

    lax.fori_loop(0, cnt, _hist, 0)
    plsc.subcore_barrier()
    pltpu.sync_copy(deg_sp.at[pl.ds(s * 640, 640)],
                    out_hbm.at[c, pl.ds(s * 640, 640)])


@functools.lru_cache(maxsize=None)
def _deg_kernel():
    return pl.kernel(
        _deg_body,
        out_type=jax.ShapeDtypeStruct((2, DEGT), jnp.float32),
        mesh=_sc_mesh(),
        compiler_params=pltpu.CompilerParams(use_tc_tiling_on_sc=False),
        scratch_types=[
            pltpu.VMEM_SHARED((DEGT,), jnp.float32),
            pltpu.VMEM((DB31, EB), jnp.int32),
            pltpu.VMEM((EB,), jnp.float32),
            pltpu.SemaphoreType.DMA,
        ],
    )


def _prop_body(t_hbm, src_hbm, dst_hbm, out_hbm, s_sp, si_v, di_v,
               rows_v, sem0, g0, g1, g2, g3, s0, s1, s2, s3):
    gs = (g0, g1, g2, g3)
    ss = (s0, s1, s2, s3)
    c = lax.axis_index("c")
    s = lax.axis_index("s")
    r0 = s * RPT
    pltpu.async_copy(t_hbm.at[c, pl.ds(r0, RPT), :],
                     s_sp.at[pl.ds(r0, RPT)], sem0).wait()
    base = s * PB
    pltpu.async_copy(src_hbm.at[pl.ds(base, PB), :],
                     si_v.at[pl.ds(0, PB)], sem0).wait()
    pltpu.async_copy(dst_hbm.at[pl.ds(base, PB), :],
                     di_v.at[pl.ds(0, PB)], sem0).wait()

    @pl.when(s == 15)
    def _():
        pltpu.async_copy(src_hbm.at[pl.ds(16 * PB, NBLK - 16 * PB), :],
                         si_v.at[pl.ds(PB, NBLK - 16 * PB)], sem0).wait()
        pltpu.async_copy(dst_hbm.at[pl.ds(16 * PB, NBLK - 16 * PB), :],
                         di_v.at[pl.ds(PB, NBLK - 16 * PB)], sem0).wait()

    plsc.subcore_barrier()

    tc_ref = t_hbm.at[c]
    for b in range(NBUF):
        pltpu.async_copy(tc_ref.at[si_v.at[b]], rows_v.at[b], gs[b])

    def outer(g, carry):
        for b in range(NBUF):
            j = g * NBUF + b
            pltpu.make_async_copy(
                tc_ref.at[si_v.at[j]], rows_v.at[b], gs[b]).wait()
            pltpu.async_copy(
                rows_v.at[b], s_sp.at[di_v.at[j]], ss[b], add=True).wait()
            pltpu.async_copy(
                tc_ref.at[si_v.at[j + NBUF]], rows_v.at[b], gs[b])
        return carry

    trips = jnp.where(s == 15, TPB15, TPB)
    lax.fori_loop(0, trips - 1, outer, 0)
    last = (trips - 1) * NBUF
    for b in range(NBUF):
        j = last + b
        pltpu.make_async_copy(tc_ref.at[si_v.at[j]], rows_v.at[b], gs[b]).wait()
        pltpu.async_copy(
            rows_v.at[b], s_sp.at[di_v.at[j]], ss[b], add=True).wait()

    plsc.subcore_barrier()
    pltpu.async_copy(s_sp.at[pl.ds(r0, RPT)],
                     out_hbm.at[c, pl.ds(r0, RPT), :], sem0).wait()


@functools.lru_cache(maxsize=None)
def _prop_kernel():
    return pl.kernel(
        _prop_body,
        out_type=jax.ShapeDtypeStruct((2, N, DH), jnp.float32),
        mesh=_sc_mesh(),
        compiler_params=pltpu.CompilerParams(use_tc_tiling_on_sc=False),
        scratch_types=[
            pltpu.VMEM_SHARED((N, DH), jnp.float32),
            pltpu.VMEM((PB + 4, EB), jnp.int32),
            pltpu.VMEM((PB + 4, EB), jnp.int32),
            pltpu.VMEM((NBUF, EB, DH), jnp.float32),
        ] + [pltpu.SemaphoreType.DMA] * (1 + 2 * NBUF),
    )


def kernel(x, edge_index, W1, b1, cw1, cw2, W2, b2):
    ei = edge_index.astype(jnp.int32)
    src2 = ei[0].reshape(NBLK, EB)
    dst2 = ei[1].reshape(NBLK, EB)

    h = pl.pallas_call(
        _mm_relu_body,
        grid=_GRID,
        in_specs=[_row_spec(D), _full_spec(D, D), _full_spec(1, D)],
        out_specs=_row_spec(D),
        out_shape=jax.ShapeDtypeStruct((N, D), jnp.float32),
    )(x, W1, b1.reshape(1, D))

    degp = _deg_kernel()(dst2, jnp.ones((EB,), jnp.float32),
                         jnp.zeros((640,), jnp.float32))
    deg2 = degp[:, :N].T

    t1 = pl.pallas_call(
        _scale_body,
        grid=_GRID,
        in_specs=[_row_spec(D), _D_SPEC],
        out_specs=_S_SPEC,
        out_shape=jax.ShapeDtypeStruct((2, N, DH), jnp.float32),
    )(h, deg2)

    s1 = _prop_kernel()(t1, src2, dst2)

    t2 = pl.pallas_call(
        _layer_body,
        grid=_GRID,
        in_specs=[_S_SPEC, _row_spec(D), _D_SPEC, _full_spec(D, D)],
        out_specs=_S_SPEC,
        out_shape=jax.ShapeDtypeStruct((2, N, DH), jnp.float32),
    )(s1, h, deg2, cw1)

    s2 = _prop_kernel()(t2, src2, dst2)

    h_out, logits = pl.pallas_call(
        _final_body,
        grid=_GRID,
        in_specs=[_S_SPEC, _row_spec(D), _D_SPEC, _full_spec(D, D),
                  _full_spec(D, DOUT), _full_spec(1, DOUT)],
        out_specs=[_row_spec(D), _row_spec(DOUT)],
        out_shape=[jax.ShapeDtypeStruct((N, D), jnp.float32),
                   jax.ShapeDtypeStruct((N, DOUT), jnp.float32)],
    )(s2, h, deg2, cw2, W2, b2.reshape(1, DOUT))

    return (h_out, logits)


# speedup vs baseline: 32.2872x; 1.0016x over previous
import functools

import jax
import jax.numpy as jnp
from jax import lax
from jax.experimental import pallas as pl
from jax.experimental.pallas import tpu as pltpu
from jax.experimental.pallas import tpu_sc as plsc

N = 10000
E = 320000
D = 128
DH = 64
DOUT = 64
ALPHA = 0.1

EB = 128
NBLK = E // EB
PB = 156
NBUF = 4
TPB = PB // NBUF
TPB15 = (NBLK - 15 * PB) // NBUF
DB = 78
DB31 = NBLK - 31 * DB
DEGT = 10240
RPT = N // 16

@functools.lru_cache(maxsize=None)
def _sc_mesh():
    return plsc.VectorSubcoreMesh(
        core_axis_name="c", subcore_axis_name="s", num_cores=2, num_subcores=16
    )


def _mm_relu_body(x_ref, w_ref, b_ref, o_ref):
    acc = jnp.dot(x_ref[...], w_ref[...], preferred_element_type=jnp.float32)
    o_ref[...] = jnp.maximum(acc + b_ref[...], 0.0)


def _scale_body(h_ref, d_ref, t_ref):
    dinv = lax.rsqrt(d_ref[:, 0] + d_ref[:, 1] + 1.0)
    t_ref[...] = h_ref[...] * dinv[:, None]


def _layer_body(s_ref, tin_ref, h0_ref, d_ref, w_ref, t_ref):
    dinv = lax.rsqrt(d_ref[:, 0] + d_ref[:, 1] + 1.0)
    st = s_ref[...] + tin_ref[...]
    u = (1.0 - ALPHA) * (st * dinv[:, None]) + ALPHA * h0_ref[...]
    h2 = jnp.dot(u, w_ref[...], preferred_element_type=jnp.float32)
    t_ref[...] = h2 * dinv[:, None]


def _final_body(s_ref, tin_ref, h0_ref, d_ref, w_ref, w2_ref, b2_ref,
                h_ref, lg_ref):
    dinv = lax.rsqrt(d_ref[:, 0] + d_ref[:, 1] + 1.0)
    st = s_ref[...] + tin_ref[...]
    u = (1.0 - ALPHA) * (st * dinv[:, None]) + ALPHA * h0_ref[...]
    h3 = jnp.dot(u, w_ref[...], preferred_element_type=jnp.float32)
    h_ref[...] = h3
    lg = jnp.dot(h3, w2_ref[...], preferred_element_type=jnp.float32)
    lg_ref[...] = lg + b2_ref[...]


_BM = 1000
_GRID = (N // _BM,)


def _row_spec(w):
    return pl.BlockSpec((_BM, w), lambda i: (i, 0))


def _full_spec(r, c):
    return pl.BlockSpec((r, c), lambda i: (0, 0))


_D_SPEC = pl.BlockSpec((_BM, 2), lambda i: (i, 0))


def _deg_body(dst_hbm, ones_hbm, zeros_hbm, out_hbm, deg_sp, idx_v, ones_v,
              sem):
    c = lax.axis_index("c")
    s = lax.axis_index("s")
    wid = c * 16 + s
    pltpu.async_copy(ones_hbm, ones_v, sem).wait()
    pltpu.sync_copy(zeros_hbm, deg_sp.at[pl.ds(s * 640, 640)])
    plsc.subcore_barrier()
    base = wid * DB
    pltpu.async_copy(dst_hbm.at[pl.ds(base, DB), :],
                     idx_v.at[pl.ds(0, DB)], sem).wait()

    @pl.when(wid == 31)
    def _():
        pltpu.async_copy(dst_hbm.at[pl.ds(32 * DB, DB31 - DB), :],
                         idx_v.at[pl.ds(DB, DB31 - DB)], sem).wait()

    cnt = jnp.where(wid == 31, DB31, DB)

    def _hist(j, carry):
        pltpu.sync_copy(ones_v, deg_sp.at[idx_v.at[j]], add=True)
        return carry

    lax.fori_loop(0, cnt, _hist, 0)
    plsc.subcore_barrier()
    pltpu.sync_copy(deg_sp.at[pl.ds(s * 640, 640)],
                    out_hbm.at[c, pl.ds(s * 640, 640)])


@functools.lru_cache(maxsize=None)
def _deg_kernel():
    return pl.kernel(
        _deg_body,
        out_type=jax.ShapeDtypeStruct((2, DEGT), jnp.float32),
        mesh=_sc_mesh(),
        compiler_params=pltpu.CompilerParams(use_tc_tiling_on_sc=False),
        scratch_types=[
            pltpu.VMEM_SHARED((DEGT,), jnp.float32),
            pltpu.VMEM((DB31, EB), jnp.int32),
            pltpu.VMEM((EB,), jnp.float32),
            pltpu.SemaphoreType.DMA,
        ],
    )


def _prop_body(t_view, zeros_hbm, se_hbm, so_hbm, dst_hbm, out_hbm, s_sp,
               si_v, di_v, rows_v, sem0, g0, g1, g2, g3, s0, s1, s2, s3):
    gs = (g0, g1, g2, g3)
    ss = (s0, s1, s2, s3)
    c = lax.axis_index("c")
    s = lax.axis_index("s")
    r0 = s * RPT
    base = s * PB

    pltpu.async_copy(zeros_hbm, s_sp.at[pl.ds(r0, RPT)], sem0).wait()

    @pl.when(c == 0)
    def _():
        pltpu.async_copy(se_hbm.at[pl.ds(base, PB), :],
                         si_v.at[pl.ds(0, PB)], sem0).wait()

    @pl.when(c == 1)
    def _():
        pltpu.async_copy(so_hbm.at[pl.ds(base, PB), :],
                         si_v.at[pl.ds(0, PB)], sem0).wait()

    pltpu.async_copy(dst_hbm.at[pl.ds(base, PB), :],
                     di_v.at[pl.ds(0, PB)], sem0).wait()

    @pl.when(jnp.logical_and(s == 15, c == 0))
    def _():
        pltpu.async_copy(se_hbm.at[pl.ds(16 * PB, NBLK - 16 * PB), :],
                         si_v.at[pl.ds(PB, NBLK - 16 * PB)], sem0).wait()

    @pl.when(jnp.logical_and(s == 15, c == 1))
    def _():
        pltpu.async_copy(so_hbm.at[pl.ds(16 * PB, NBLK - 16 * PB), :],
                         si_v.at[pl.ds(PB, NBLK - 16 * PB)], sem0).wait()

    @pl.when(s == 15)
    def _():
        pltpu.async_copy(dst_hbm.at[pl.ds(16 * PB, NBLK - 16 * PB), :],
                         di_v.at[pl.ds(PB, NBLK - 16 * PB)], sem0).wait()

    plsc.subcore_barrier()

    for b in range(NBUF):
        pltpu.async_copy(t_view.at[si_v.at[b]], rows_v.at[b], gs[b])

    def outer(g, carry):
        for b in range(NBUF):
            j = g * NBUF + b
            pltpu.make_async_copy(
                t_view.at[si_v.at[j]], rows_v.at[b], gs[b]).wait()
            pltpu.async_copy(
                rows_v.at[b], s_sp.at[di_v.at[j]], ss[b], add=True).wait()
            pltpu.async_copy(
                t_view.at[si_v.at[j + NBUF]], rows_v.at[b], gs[b])
        return carry

    trips = jnp.where(s == 15, TPB15, TPB)
    lax.fori_loop(0, trips - 1, outer, 0)
    last = (trips - 1) * NBUF
    for b in range(NBUF):
        j = last + b
        pltpu.make_async_copy(t_view.at[si_v.at[j]], rows_v.at[b],
                              gs[b]).wait()
        pltpu.async_copy(
            rows_v.at[b], s_sp.at[di_v.at[j]], ss[b], add=True).wait()

    plsc.subcore_barrier()

    @pl.when(c == 0)
    def _():
        pltpu.async_copy(s_sp.at[pl.ds(r0, RPT)],
                         out_hbm.at[pl.ds(r0, RPT), pl.ds(0, DH)], sem0).wait()

    @pl.when(c == 1)
    def _():
        pltpu.async_copy(s_sp.at[pl.ds(r0, RPT)],
                         out_hbm.at[pl.ds(r0, RPT), pl.ds(DH, DH)],
                         sem0).wait()


@functools.lru_cache(maxsize=None)
def _prop_kernel():
    return pl.kernel(
        _prop_body,
        out_type=jax.ShapeDtypeStruct((N, D), jnp.float32),
        mesh=_sc_mesh(),
        compiler_params=pltpu.CompilerParams(use_tc_tiling_on_sc=False),
        scratch_types=[
            pltpu.VMEM_SHARED((N, DH), jnp.float32),
            pltpu.VMEM((PB + 4, EB), jnp.int32),
            pltpu.VMEM((PB + 4, EB), jnp.int32),
            pltpu.VMEM((NBUF, EB, DH), jnp.float32),
        ] + [pltpu.SemaphoreType.DMA] * (1 + 2 * NBUF),
    )


def kernel(x, edge_index, W1, b1, cw1, cw2, W2, b2):
    ei = edge_index.astype(jnp.int32)
    src2 = ei[0].reshape(NBLK, EB)
    dst2 = ei[1].reshape(NBLK, EB)
    src_even = src2 * 2
    src_odd = src_even + 1

    h = pl.pallas_call(
        _mm_relu_body,
        grid=_GRID,
        in_specs=[_row_spec(D), _full_spec(D, D), _full_spec(1, D)],
        out_specs=_row_spec(D),
        out_shape=jax.ShapeDtypeStruct((N, D), jnp.float32),
    )(x, W1, b1.reshape(1, D))

    degp = _deg_kernel()(dst2, jnp.ones((EB,), jnp.float32),
                         jnp.zeros((640,), jnp.float32))
    deg2 = degp[:, :N].T

    t1 = pl.pallas_call(
        _scale_body,
        grid=_GRID,
        in_specs=[_row_spec(D), _D_SPEC],
        out_specs=_row_spec(D),
        out_shape=jax.ShapeDtypeStruct((N, D), jnp.float32),
    )(h, deg2)

    zrows = jnp.zeros((RPT, DH), jnp.float32)
    t1v = lax.optimization_barrier(t1.reshape(2 * N, DH))
    s1 = _prop_kernel()(t1v, zrows, src_even, src_odd, dst2)

    t2 = pl.pallas_call(
        _layer_body,
        grid=_GRID,
        in_specs=[_row_spec(D), _row_spec(D), _row_spec(D), _D_SPEC,
                  _full_spec(D, D)],
        out_specs=_row_spec(D),
        out_shape=jax.ShapeDtypeStruct((N, D), jnp.float32),
    )(s1, t1, h, deg2, cw1)

    t2v = lax.optimization_barrier(t2.reshape(2 * N, DH))
    s2 = _prop_kernel()(t2v, zrows, src_even, src_odd, dst2)

    h_out, logits = pl.pallas_call(
        _final_body,
        grid=_GRID,
        in_specs=[_row_spec(D), _row_spec(D), _row_spec(D), _D_SPEC,
                  _full_spec(D, D), _full_spec(D, DOUT), _full_spec(1, DOUT)],
        out_specs=[_row_spec(D), _row_spec(DOUT)],
        out_shape=[jax.ShapeDtypeStruct((N, D), jnp.float32),
                   jax.ShapeDtypeStruct((N, DOUT), jnp.float32)],
    )(s2, t2, h, deg2, cw2, W2, b2.reshape(1, DOUT))

    return (h_out, logits)
